# Initial kernel scaffold; baseline (speedup 1.0000x reference)
import functools

import jax
import jax.numpy as jnp
from jax import lax
from jax.experimental import pallas as pl
from jax.experimental.pallas import tpu as pltpu
from jax.experimental.pallas import tpu_sc as plsc

D = 64
V = 1024

_SCORE_PREC = lax.Precision.DEFAULT
_MM_PREC = lax.Precision.DEFAULT


def _argmin_body(ng, w_ref, rs_ref, cbt_ref, idx_ref):
    x = w_ref[...] / rs_ref[...]
    cbt = cbt_ref[...]
    c2 = jnp.sum(cbt * cbt, axis=0, keepdims=True)
    br = x.shape[0]
    iota = lax.broadcasted_iota(jnp.int32, (br, V), 1)
    cols = []
    for g in range(ng):
        xg = x[:, D * g:D * (g + 1)]
        gmat = lax.dot_general(xg, cbt, (((1,), (0,)), ((), ())),
                               precision=_SCORE_PREC)
        x2 = jnp.sum(xg * xg, axis=1, keepdims=True)
        d2 = x2 - 2.0 * gmat + c2
        m = jnp.min(d2, axis=1, keepdims=True)
        first = jnp.min(jnp.where(d2 == m, iota, V), axis=1)
        cols.append(first)
    idx_ref[...] = jnp.stack(cols, axis=1)


def _pq_argmin(w, rs, cbt, block_rows):
    n, width = w.shape
    ng = width // D
    grid = (n // block_rows,)
    return pl.pallas_call(
        functools.partial(_argmin_body, ng),
        grid=grid,
        in_specs=[
            pl.BlockSpec((block_rows, width), lambda i: (i, 0)),
            pl.BlockSpec((block_rows, 1), lambda i: (i, 0)),
            pl.BlockSpec((D, V), lambda i: (0, 0)),
        ],
        out_specs=pl.BlockSpec((block_rows, ng), lambda i: (i, 0)),
        out_shape=jax.ShapeDtypeStruct((n, ng), jnp.int32),
    )(w, rs, cbt)


_CH = 128


def _sc_gather(table, idx):
    n = idx.shape[0]
    nw = 32
    b_per_w = n // nw
    n_ch = b_per_w // _CH
    mesh = plsc.VectorSubcoreMesh(core_axis_name="c", subcore_axis_name="s")

    @functools.partial(
        pl.kernel,
        out_type=jax.ShapeDtypeStruct((n, D), jnp.float32),
        mesh=mesh,
        scratch_types=[
            pltpu.VMEM((_CH,), jnp.int32),
            pltpu.VMEM((_CH, D), jnp.float32),
            pltpu.SemaphoreType.DMA,
        ],
    )
    def k(table_hbm, idx_hbm, out_hbm, idx_v, rows_v, sem):
        wid = lax.axis_index("s") * 2 + lax.axis_index("c")
        base = wid * b_per_w

        def body(j, carry):
            off = base + j * _CH
            pltpu.sync_copy(idx_hbm.at[pl.ds(off, _CH)], idx_v)
            pltpu.async_copy(table_hbm.at[idx_v], rows_v, sem).wait()
            pltpu.sync_copy(rows_v, out_hbm.at[pl.ds(off, _CH)])
            return carry

        lax.fori_loop(0, n_ch, body, 0)

    return k(table, idx)


def _mm_body(uq_ref, rsu_ref, bq_ref, rsb_ref, out_ref):
    a = uq_ref[...] * rsu_ref[...]
    b = bq_ref[...] * rsb_ref[...]
    out_ref[...] = lax.dot_general(a, b, (((1,), (0,)), ((), ())),
                                   precision=_MM_PREC)


def _scaled_matmul(uq, rs_u, bq, rs_b, block_rows):
    n, k = uq.shape
    m = bq.shape[1]
    grid = (n // block_rows,)
    return pl.pallas_call(
        _mm_body,
        grid=grid,
        in_specs=[
            pl.BlockSpec((block_rows, k), lambda i: (i, 0)),
            pl.BlockSpec((block_rows, 1), lambda i: (i, 0)),
            pl.BlockSpec((k, m), lambda i: (0, 0)),
            pl.BlockSpec((k, 1), lambda i: (0, 0)),
        ],
        out_specs=pl.BlockSpec((block_rows, m), lambda i: (i, 0)),
        out_shape=jax.ShapeDtypeStruct((n, m), jnp.float32),
    )(uq, rs_u, bq, rs_b)


def kernel(U, B, rs_U, rs_B, cb_U, cb_B):
    cbt_U = cb_U.T
    cbt_B = cb_B.T
    idx_U = _pq_argmin(U, rs_U, cbt_U, block_rows=512)
    idx_B = _pq_argmin(B, rs_B, cbt_B, block_rows=256)
    qU = _sc_gather(cb_U, idx_U.reshape(-1))
    qB = _sc_gather(cb_B, idx_B.reshape(-1))
    Uq = qU.reshape(16384, 256)
    Bq = qB.reshape(256, 1024)
    return _scaled_matmul(Uq, rs_U, Bq, rs_B, block_rows=512)


# speedup vs baseline: 1.6967x; 1.6967x over previous
import functools

import jax
import jax.numpy as jnp
from jax import lax
from jax.experimental import pallas as pl
from jax.experimental.pallas import tpu as pltpu
from jax.experimental.pallas import tpu_sc as plsc

D = 64
V = 1024

_SCORE_PREC = lax.Precision.DEFAULT
_MM_PREC = lax.Precision.DEFAULT


def _argmin_body(ng, w_ref, rs_ref, cbt_ref, idx_ref):
    x = w_ref[...] / rs_ref[...]
    cbt = cbt_ref[...]
    c2 = jnp.sum(cbt * cbt, axis=0, keepdims=True)
    br = x.shape[0]
    iota = lax.broadcasted_iota(jnp.int32, (br, V), 1)
    cols = []
    for g in range(ng):
        xg = x[:, D * g:D * (g + 1)]
        gmat = lax.dot_general(xg, cbt, (((1,), (0,)), ((), ())),
                               precision=_SCORE_PREC)
        x2 = jnp.sum(xg * xg, axis=1, keepdims=True)
        d2 = x2 - 2.0 * gmat + c2
        m = jnp.min(d2, axis=1, keepdims=True)
        first = jnp.min(jnp.where(d2 == m, iota, V), axis=1)
        cols.append(first)
    idx_ref[...] = jnp.stack(cols, axis=1)


def _pq_argmin(w, rs, cbt, block_rows):
    n, width = w.shape
    ng = width // D
    grid = (n // block_rows,)
    return pl.pallas_call(
        functools.partial(_argmin_body, ng),
        grid=grid,
        in_specs=[
            pl.BlockSpec((block_rows, width), lambda i: (i, 0)),
            pl.BlockSpec((block_rows, 1), lambda i: (i, 0)),
            pl.BlockSpec((D, V), lambda i: (0, 0)),
        ],
        out_specs=pl.BlockSpec((block_rows, ng), lambda i: (i, 0)),
        out_shape=jax.ShapeDtypeStruct((n, ng), jnp.int32),
    )(w, rs, cbt)


_CH = 128


def _sc_gather(table, idx):
    n = idx.shape[0]
    nw = 32
    b_per_w = n // nw
    n_ch = b_per_w // _CH
    mesh = plsc.VectorSubcoreMesh(core_axis_name="c", subcore_axis_name="s")

    @functools.partial(
        pl.kernel,
        out_type=jax.ShapeDtypeStruct((n, D), jnp.float32),
        mesh=mesh,
        compiler_params=pltpu.CompilerParams(use_tc_tiling_on_sc=False),
        scratch_types=[
            pltpu.VMEM((_CH,), jnp.int32),
            pltpu.VMEM((_CH, D), jnp.float32),
            pltpu.SemaphoreType.DMA,
        ],
    )
    def k(table_hbm, idx_hbm, out_hbm, idx_v, rows_v, sem):
        wid = lax.axis_index("s") * 2 + lax.axis_index("c")
        base = wid * b_per_w

        def body(j, carry):
            off = base + j * _CH
            pltpu.sync_copy(idx_hbm.at[pl.ds(off, _CH)], idx_v)
            pltpu.async_copy(table_hbm.at[idx_v], rows_v, sem).wait()
            pltpu.sync_copy(rows_v, out_hbm.at[pl.ds(off, _CH)])
            return carry

        lax.fori_loop(0, n_ch, body, 0)

    return k(table, idx)


def _mm_body(uq_ref, rsu_ref, bq_ref, rsb_ref, out_ref):
    a = uq_ref[...] * rsu_ref[...]
    b = bq_ref[...] * rsb_ref[...]
    out_ref[...] = lax.dot_general(a, b, (((1,), (0,)), ((), ())),
                                   precision=_MM_PREC)


def _scaled_matmul(uq, rs_u, bq, rs_b, block_rows):
    n, k = uq.shape
    m = bq.shape[1]
    grid = (n // block_rows,)
    return pl.pallas_call(
        _mm_body,
        grid=grid,
        in_specs=[
            pl.BlockSpec((block_rows, k), lambda i: (i, 0)),
            pl.BlockSpec((block_rows, 1), lambda i: (i, 0)),
            pl.BlockSpec((k, m), lambda i: (0, 0)),
            pl.BlockSpec((k, 1), lambda i: (0, 0)),
        ],
        out_specs=pl.BlockSpec((block_rows, m), lambda i: (i, 0)),
        out_shape=jax.ShapeDtypeStruct((n, m), jnp.float32),
    )(uq, rs_u, bq, rs_b)


def kernel(U, B, rs_U, rs_B, cb_U, cb_B):
    cbt_U = cb_U.T
    cbt_B = cb_B.T
    idx_U = _pq_argmin(U, rs_U, cbt_U, block_rows=512)
    idx_B = _pq_argmin(B, rs_B, cbt_B, block_rows=256)
    qU = _sc_gather(cb_U, idx_U.reshape(-1))
    qB = _sc_gather(cb_B, idx_B.reshape(-1))
    Uq = qU.reshape(16384, 256)
    Bq = qB.reshape(256, 1024)
    return _scaled_matmul(Uq, rs_U, Bq, rs_B, block_rows=512)


# speedup vs baseline: 2.0854x; 1.2291x over previous
import functools

import jax
import jax.numpy as jnp
from jax import lax
from jax.experimental import pallas as pl
from jax.experimental.pallas import tpu as pltpu
from jax.experimental.pallas import tpu_sc as plsc

D = 64
V = 1024

_SCORE_PREC = lax.Precision.DEFAULT
_MM_PREC = lax.Precision.DEFAULT


_CHL = 128


def _argmin_body(ng, w_ref, rs_ref, cbt_ref, idx_ref):
    x = w_ref[...] / rs_ref[...]
    cbt = cbt_ref[...]
    c2h = 0.5 * jnp.sum(cbt * cbt, axis=0, keepdims=True)
    br = x.shape[0]
    iota = lax.broadcasted_iota(jnp.int32, (br, V), 1).astype(jnp.float32)
    cols = []
    for g in range(ng):
        xg = x[:, D * g:D * (g + 1)]
        gmat = lax.dot_general(xg, cbt, (((1,), (0,)), ((), ())),
                               precision=_SCORE_PREC)
        t = c2h - gmat
        m = jnp.min(t, axis=1, keepdims=True)
        first = jnp.min(jnp.where(t == m, iota, jnp.float32(V)), axis=1)
        cols.append(first.astype(jnp.int32))
    idx_ref[...] = jnp.stack(cols, axis=1)


def _pq_argmin(w, rs, cbt, block_rows):
    n, width = w.shape
    ng = width // D
    grid = (n // block_rows,)
    return pl.pallas_call(
        functools.partial(_argmin_body, ng),
        grid=grid,
        in_specs=[
            pl.BlockSpec((block_rows, width), lambda i: (i, 0)),
            pl.BlockSpec((block_rows, 1), lambda i: (i, 0)),
            pl.BlockSpec((D, V), lambda i: (0, 0)),
        ],
        out_specs=pl.BlockSpec((block_rows, ng), lambda i: (i, 0)),
        out_shape=jax.ShapeDtypeStruct((n, ng), jnp.int32),
    )(w, rs, cbt)


_CH = 128


def _sc_gather(table, idx):
    n = idx.shape[0]
    nw = 32
    b_per_w = n // nw
    n_ch = b_per_w // _CH
    mesh = plsc.VectorSubcoreMesh(core_axis_name="c", subcore_axis_name="s")

    @functools.partial(
        pl.kernel,
        out_type=jax.ShapeDtypeStruct((n, D), jnp.float32),
        mesh=mesh,
        compiler_params=pltpu.CompilerParams(use_tc_tiling_on_sc=False),
        scratch_types=[
            pltpu.VMEM((_CH,), jnp.int32),
            pltpu.VMEM((_CH, D), jnp.float32),
            pltpu.SemaphoreType.DMA,
        ],
    )
    def k(table_hbm, idx_hbm, out_hbm, idx_v, rows_v, sem):
        wid = lax.axis_index("s") * 2 + lax.axis_index("c")
        base = wid * b_per_w

        def body(j, carry):
            off = base + j * _CH
            pltpu.sync_copy(idx_hbm.at[pl.ds(off, _CH)], idx_v)
            pltpu.async_copy(table_hbm.at[idx_v], rows_v, sem).wait()
            pltpu.sync_copy(rows_v, out_hbm.at[pl.ds(off, _CH)])
            return carry

        lax.fori_loop(0, n_ch, body, 0)

    return k(table, idx)


def _mm_body(uq_ref, rsu_ref, bq_ref, rsb_ref, out_ref):
    a = uq_ref[...] * rsu_ref[...]
    b = bq_ref[...] * rsb_ref[...]
    out_ref[...] = lax.dot_general(a, b, (((1,), (0,)), ((), ())),
                                   precision=_MM_PREC)


def _scaled_matmul(uq, rs_u, bq, rs_b, block_rows):
    n, k = uq.shape
    m = bq.shape[1]
    grid = (n // block_rows,)
    return pl.pallas_call(
        _mm_body,
        grid=grid,
        in_specs=[
            pl.BlockSpec((block_rows, k), lambda i: (i, 0)),
            pl.BlockSpec((block_rows, 1), lambda i: (i, 0)),
            pl.BlockSpec((k, m), lambda i: (0, 0)),
            pl.BlockSpec((k, 1), lambda i: (0, 0)),
        ],
        out_specs=pl.BlockSpec((block_rows, m), lambda i: (i, 0)),
        out_shape=jax.ShapeDtypeStruct((n, m), jnp.float32),
    )(uq, rs_u, bq, rs_b)


def kernel(U, B, rs_U, rs_B, cb_U, cb_B):
    cbt_U = cb_U.T
    cbt_B = cb_B.T
    idx_U = _pq_argmin(U, rs_U, cbt_U, block_rows=512)
    idx_B = _pq_argmin(B, rs_B, cbt_B, block_rows=256)
    qU = _sc_gather(cb_U, idx_U.reshape(-1))
    qB = _sc_gather(cb_B, idx_B.reshape(-1))
    Uq = qU.reshape(16384, 256)
    Bq = qB.reshape(256, 1024)
    return _scaled_matmul(Uq, rs_U, Bq, rs_B, block_rows=1024)
